# baseline (speedup 1.0000x reference)
import functools
import jax
import jax.numpy as jnp
from jax import lax
from jax.experimental import pallas as pl
from jax.experimental.pallas import tpu as pltpu
from jax.experimental.pallas import tpu_sc as plsc

B, NV, NE, D = 4, 10000, 160000, 128
NC, NS, L = 2, 16, 16
NW = NC * NS
EPW = NE // NW
KW = 160
NBLK = EPW // KW
KT = EPW - NBLK * KW
NIDX = 2 * EPW
CP_ROWS = 1248
TL_ROWS = 640
TL_LAST = NV - 15 * TL_ROWS
SP = 40

_mesh = plsc.VectorSubcoreMesh(core_axis_name="c", subcore_axis_name="s")


@functools.partial(
    pl.kernel,
    out_type=jax.ShapeDtypeStruct((B, NV + NE, D), jnp.float32),
    mesh=_mesh,
    scratch_types=[
        pltpu.VMEM_SHARED((NV, D), jnp.float32),
        pltpu.VMEM((NIDX,), jnp.int32),
        pltpu.VMEM((KW, D), jnp.float32),
        pltpu.VMEM((KW, D), jnp.float32),
        pltpu.SemaphoreType.DMA,
        pltpu.SemaphoreType.DMA,
        pltpu.SemaphoreType.DMA,
        pltpu.SemaphoreType.DMA,
        pltpu.SemaphoreType.DMA,
    ],
)
def _unpool_kernel(vflat, ic, out,
                   table, idxall, r0, r1,
                   sg0, sg1, sw0, sw1, sc):
    rows = [r0, r1]
    semG = [sg0, sg1]
    semW = [sw0, sw1]

    cid = lax.axis_index("c")
    sid = lax.axis_index("s")
    wid = sid * NC + cid

    cb = wid // 8
    cr0 = (wid % 8) * CP_ROWS
    cp = pltpu.async_copy(vflat.at[pl.ds(cb * NV + cr0, CP_ROWS)],
                          out.at[cb, pl.ds(cr0, CP_ROWS)], sc)
    RREM = NV - 8 * CP_ROWS

    @pl.when(wid < B)
    def _():
        pltpu.async_copy(vflat.at[pl.ds(wid * NV + 8 * CP_ROWS, RREM)],
                         out.at[wid, pl.ds(8 * CP_ROWS, RREM)], sc)

    pltpu.sync_copy(ic.at[pl.ds(wid * NIDX, NIDX)], idxall)
    row0 = wid * EPW

    def i0_ref(w, n):
        return idxall.at[pl.ds(w * 2 * KW, n)]

    def i1_ref(w, n):
        return idxall.at[pl.ds(w * 2 * KW + n, n)]

    def dst(p, n):
        return rows[p] if n == KW else rows[p].at[pl.ds(0, n)]

    def fire_g1(p, w, n=KW):
        pltpu.async_copy(table.at[i0_ref(w, n)], dst(p, n), semG[p])

    def wait_g1(p, w, n=KW):
        pltpu.make_async_copy(table.at[i0_ref(w, n)], dst(p, n),
                              semG[p]).wait()

    def fire_g2(p, w, n=KW):
        pltpu.async_copy(table.at[i1_ref(w, n)], dst(p, n), semG[p], add=True)

    def wait_g2(p, w, n=KW):
        pltpu.make_async_copy(table.at[i1_ref(w, n)], dst(p, n),
                              semG[p]).wait()

    def wait_write(p, n=KW):
        pltpu.make_async_copy(dst(p, n), out.at[0, pl.ds(NV, n)],
                              semW[p]).wait()

    def fire_write(p, b, w, n=KW):
        pltpu.async_copy(dst(p, n),
                         out.at[b, pl.ds(NV + row0 + w * KW, n)], semW[p])

    for b in range(B):
        def scale_piece(r_off):
            stg = rows[0].at[pl.ds(0, SP)]
            pltpu.sync_copy(vflat.at[pl.ds(b * NV + r_off, SP)], stg)

            @plsc.parallel_loop(0, SP, unroll=2)
            def _(r):
                for j in range(D // L):
                    sl = pl.ds(j * L, L)
                    rows[0][r, sl] = rows[0][r, sl] * 0.5

            pltpu.sync_copy(stg, table.at[pl.ds(r_off, SP)])

        @pl.when(sid < NS - 1)
        def _():
            def pbody(i, carry):
                scale_piece(sid * TL_ROWS + i * SP)
                return carry

            lax.fori_loop(0, TL_ROWS // SP, pbody, 0)

        @pl.when(sid == NS - 1)
        def _():
            def pbody(i, carry):
                scale_piece(15 * TL_ROWS + i * SP)
                return carry

            lax.fori_loop(0, TL_LAST // SP, pbody, 0)

        plsc.subcore_barrier()

        fire_g1(0, 0)

        def pair_body(g, carry):
            for p in (0, 1):
                w = 2 * g + p
                q = p ^ 1
                wait_g1(p, w)
                fire_g2(p, w)

                @pl.when(w >= 1)
                def _():
                    wait_write(q)

                fire_g1(q, w + 1)
                wait_g2(p, w)
                fire_write(p, b, w)
            return carry

        lax.fori_loop(0, NBLK // 2, pair_body, 0)
        wait_g1(0, NBLK - 1)
        fire_g2(0, NBLK - 1)
        wait_write(1)
        fire_g1(1, NBLK, KT)
        wait_g2(0, NBLK - 1)
        fire_write(0, b, NBLK - 1)
        wait_g1(1, NBLK, KT)
        fire_g2(1, NBLK, KT)
        wait_g2(1, NBLK, KT)
        fire_write(1, b, NBLK, KT)

        wait_write(0)
        wait_write(1, KT)
        plsc.subcore_barrier()

    cp.wait()

    @pl.when(wid < B)
    def _():
        pltpu.make_async_copy(vflat.at[pl.ds(wid * NV + 8 * CP_ROWS, RREM)],
                              out.at[wid, pl.ds(8 * CP_ROWS, RREM)], sc).wait()


def kernel(vertices, unpool_idx):
    vflat = vertices.reshape(B * NV, D)
    e = unpool_idx.reshape(NW, EPW, 2)
    full = e[:, :NBLK * KW, :].reshape(NW, NBLK, KW, 2)
    full = full.transpose(0, 1, 3, 2).reshape(NW, NBLK * 2 * KW)
    tail = e[:, NBLK * KW:, :].transpose(0, 2, 1).reshape(NW, 2 * KT)
    ic = jnp.concatenate([full, tail], axis=1).reshape(-1)
    return _unpool_kernel(vflat, ic)
